# baseline (speedup 1.0000x reference)
import functools
import numpy as np
import jax
import jax.numpy as jnp
from jax import lax
from jax.experimental import pallas as pl
from jax.experimental.pallas import tpu as pltpu
from jax.experimental.pallas import tpu_sc as plsc

D = 64
NV = 50000
NED = 400000
NRELX = 401
NC, NS = 2, 16

CH = 256
EPT = 25600
NCHUNK = EPT // CH
EPAD = EPT * NS
ZROWS = 50048
TPR = ZROWS // NS

HR = 3200


def _build_spectral_mats():
    F = np.zeros((D, D), dtype=np.float64)
    for i in range(D):
        e = np.zeros(D)
        e[i] = 1.0
        fa = np.fft.rfft(e)
        F[i, 0:32] = fa.real[0:32]
        F[i, 32] = fa.real[32]
        F[i, 33:48] = fa.imag[1:16]
        F[i, 48:64] = fa.imag[16:32]
    Fi = np.zeros((D, D), dtype=np.float64)
    for p in range(D):
        z = np.zeros(D // 2 + 1, dtype=complex)
        if p <= 32:
            z[p] += 1.0
        else:
            z[p - 32] += 1j
        Fi[p] = np.fft.irfft(z, n=D)
    return F.astype(np.float32), Fi.astype(np.float32)


_F_NP, _FI_NP = _build_spectral_mats()
_LANE0_NP = (np.arange(16) == 0)


def _deg_body(rows_hbm, deg_out, rowv, hist_v):
    c = lax.axis_index("c")
    s = lax.axis_index("s")
    zero16 = jnp.zeros((16,), jnp.float32)

    def zero_row(i, carry):
        hist_v[i, :] = zero16
        return carry
    lax.fori_loop(0, HR, zero_row, 0)

    ccal, _ = plsc.scan_count(jnp.zeros((16,), jnp.int32))
    off = jnp.int32(16) - jnp.max(ccal)

    ebase = s * EPT

    def chunk(k, carry):
        pltpu.sync_copy(rows_hbm.at[c, pl.ds(ebase + k * CH, CH)], rowv)

        def vreg(q, carry2):
            ridx = rowv[pl.ds(q * 16, 16)]
            cnt, lastm = plsc.scan_count(ridx)
            val = (cnt + off).astype(jnp.float32)
            hi = lax.shift_right_logical(ridx, 4)
            lo = lax.bitwise_and(ridx, 15)
            plsc.addupdate_scatter(hist_v, [hi, lo], val, mask=lastm)
            return carry2
        lax.fori_loop(0, CH // 16, vreg, 0)
        return carry
    lax.fori_loop(0, NCHUNK, chunk, 0)

    pltpu.sync_copy(hist_v, deg_out.at[c, s])


def _quarter(q, arows, btv, i, t, lane0):
    if q == 0:
        a0 = arows[i, pl.ds(0, 16)]
        a2 = arows[i, pl.ds(32, 16)]
        b0 = btv[t, pl.ds(0, 16)]
        b2 = btv[t, pl.ds(32, 16)]
        return a0 * b0 + jnp.where(lane0, jnp.float32(0.0), a2 * b2)
    if q == 1:
        a1 = arows[i, pl.ds(16, 16)]
        a3 = arows[i, pl.ds(48, 16)]
        b1 = btv[t, pl.ds(16, 16)]
        b3 = btv[t, pl.ds(48, 16)]
        return a1 * b1 + a3 * b3
    if q == 2:
        a0 = arows[i, pl.ds(0, 16)]
        a2 = arows[i, pl.ds(32, 16)]
        b0 = btv[t, pl.ds(0, 16)]
        b2 = btv[t, pl.ds(32, 16)]
        return jnp.where(lane0, a2 * b2, a0 * b2 - a2 * b0)
    a1 = arows[i, pl.ds(16, 16)]
    a3 = arows[i, pl.ds(48, 16)]
    b1 = btv[t, pl.ds(16, 16)]
    b3 = btv[t, pl.ds(48, 16)]
    return a1 * b3 - a3 * b1


def _make_edge_body(q_core0, q_core1):
    def _edge_body(ep_hbm, a_hbm, b_hbm, z_out,
                   idxb, rowsb, ar0, ar1, zb0, zb1, btv, zerob, spmem_z,
                   sg0, sg1, ss0, ss1, si0, si1):
        c = lax.axis_index("c")
        s = lax.axis_index("s")
        pltpu.sync_copy(b_hbm, btv)

        zero16 = jnp.zeros((16,), jnp.float32)

        def zero_zb(i, carry):
            zerob[i, :] = zero16
            return carry
        lax.fori_loop(0, TPR // 8, zero_zb, 0)

        base = s * TPR
        for j in range(8):
            pltpu.sync_copy(zerob, spmem_z.at[pl.ds(base + j * (TPR // 8),
                                                    TPR // 8)])
        plsc.subcore_barrier()

        lane0 = lax.iota(jnp.int32, 16) == 0
        cbase = s * NCHUNK
        bufs = [(0, ar0, zb0, sg0, ss0, si0), (1, ar1, zb1, sg1, ss1, si1)]

        def compute_chunk(b, arows, zbuf):
            def group(q):
                def body(g, carry2):
                    et16 = idxb[b * 3 + 2, pl.ds(g * 16, 16)]
                    for j in range(16):
                        i = g * 16 + j
                        t = et16[j]
                        zbuf[i, :] = _quarter(q, arows, btv, i, t, lane0)
                    return carry2
                return body

            @pl.when(c == 0)
            def _():
                lax.fori_loop(0, CH // 16, group(q_core0), 0)

            @pl.when(c == 1)
            def _():
                lax.fori_loop(0, CH // 16, group(q_core1), 0)

        pltpu.sync_copy(ep_hbm.at[cbase], idxb.at[pl.ds(0, 3)])
        pltpu.async_copy(a_hbm.at[idxb.at[0]], ar0, sg0)
        pltpu.async_copy(ep_hbm.at[cbase + 1], idxb.at[pl.ds(3, 3)], si1)

        def pair(kk, carry):
            for (b, arows, zbuf, sg, ss, si) in bufs:
                k = kk * 2 + b
                nb, narows, _, nsg, _, nsi = bufs[1 - b]

                @pl.when(k >= 2)
                def _():
                    pltpu.make_async_copy(
                        zbuf, spmem_z.at[rowsb.at[b]], ss).wait()
                for g in range(CH // 16):
                    rowsb[b, pl.ds(g * 16, 16)] = \
                        idxb[b * 3 + 1, pl.ds(g * 16, 16)]

                @pl.when(k + 1 < NCHUNK)
                def _():
                    pltpu.make_async_copy(
                        ep_hbm.at[cbase], idxb.at[pl.ds(nb * 3, 3)],
                        nsi).wait()
                    pltpu.async_copy(a_hbm.at[idxb.at[nb * 3]], narows, nsg)

                pltpu.make_async_copy(a_hbm.at[pl.ds(0, CH)], arows, sg).wait()
                compute_chunk(b, arows, zbuf)
                pltpu.async_copy(
                    zbuf, spmem_z.at[rowsb.at[b]], ss, add=True)

                @pl.when(k + 2 < NCHUNK)
                def _():
                    pltpu.async_copy(ep_hbm.at[cbase + k + 2],
                                     idxb.at[pl.ds(b * 3, 3)], si)
            return carry
        lax.fori_loop(0, NCHUNK // 2, pair, 0)

        pltpu.make_async_copy(zb0, spmem_z.at[rowsb.at[0]], ss0).wait()
        pltpu.make_async_copy(zb1, spmem_z.at[rowsb.at[1]], ss1).wait()

        plsc.subcore_barrier()
        pltpu.sync_copy(spmem_z.at[pl.ds(base, TPR)],
                        z_out.at[c, pl.ds(base, TPR)])
    return _edge_body


def _make_sc_kernels():
    mesh = plsc.VectorSubcoreMesh(core_axis_name="c", subcore_axis_name="s")
    deg_k = pl.kernel(
        _deg_body,
        out_type=jax.ShapeDtypeStruct((NC, NS, HR, 16), jnp.float32),
        mesh=mesh,
        scratch_types=[
            pltpu.VMEM((CH,), jnp.int32),
            pltpu.VMEM((HR, 16), jnp.float32),
        ],
        compiler_params=pltpu.CompilerParams(
            needs_layout_passes=False, use_tc_tiling_on_sc=False),
    )
    def make_edge(q_core0, q_core1):
        return pl.kernel(
            _make_edge_body(q_core0, q_core1),
            out_type=jax.ShapeDtypeStruct((NC, ZROWS, 16), jnp.float32),
            mesh=mesh,
            scratch_types=[
                pltpu.VMEM((6, CH), jnp.int32),
                pltpu.VMEM((2, CH), jnp.int32),
                pltpu.VMEM((CH, D), jnp.float32),
                pltpu.VMEM((CH, D), jnp.float32),
                pltpu.VMEM((CH, 16), jnp.float32),
                pltpu.VMEM((CH, 16), jnp.float32),
                pltpu.VMEM((NRELX, D), jnp.float32),
                pltpu.VMEM((TPR // 8, 16), jnp.float32),
                pltpu.VMEM_SHARED((ZROWS, 16), jnp.float32),
                pltpu.SemaphoreType.DMA,
                pltpu.SemaphoreType.DMA,
                pltpu.SemaphoreType.DMA,
                pltpu.SemaphoreType.DMA,
                pltpu.SemaphoreType.DMA,
                pltpu.SemaphoreType.DMA,
            ],
            compiler_params=pltpu.CompilerParams(use_tc_tiling_on_sc=False),
        )
    return deg_k, make_edge(0, 2), make_edge(1, 3)


def _params_body(relx_ref, w_in_ref, w_out_ref, w_loop_ref, w_rel_ref,
                 f_ref, fi_ref,
                 bt_ref, gin_ref, gout_ref, gloop_ref, relw_ref):
    relx = relx_ref[...]
    fi = fi_ref[...]
    bt_ref[...] = jnp.dot(relx, f_ref[...], preferred_element_type=jnp.float32, precision=lax.Precision.HIGHEST)
    gin_ref[...] = jnp.dot(fi, w_in_ref[...], preferred_element_type=jnp.float32, precision=lax.Precision.HIGHEST)
    gout_ref[...] = jnp.dot(fi, w_out_ref[...], preferred_element_type=jnp.float32, precision=lax.Precision.HIGHEST)
    gloop_ref[...] = jnp.dot(fi, w_loop_ref[...], preferred_element_type=jnp.float32, precision=lax.Precision.HIGHEST)
    relw_ref[...] = jnp.dot(relx, w_rel_ref[...], preferred_element_type=jnp.float32, precision=lax.Precision.HIGHEST)


def _dinv(deg):
    return jnp.where(deg > 0, lax.rsqrt(jnp.maximum(deg, 1e-30)), 0.0)


def _degred_body(dt_ref, deg_ref):
    deg_ref[...] = jnp.sum(dt_ref[...], axis=1)


def _prep_body(x_ref, degi_ref, dego_ref, f_ref, xp_ref, ain_ref, aout_ref):
    xp = jnp.dot(x_ref[...], f_ref[...], preferred_element_type=jnp.float32, precision=lax.Precision.HIGHEST)
    xp_ref[...] = xp
    ain_ref[...] = _dinv(degi_ref[...]) * xp
    aout_ref[...] = _dinv(dego_ref[...]) * xp


def _combine_body(zin_ref, zout_ref, xp_ref, degi_ref, dego_ref, bl_ref,
                  gin_ref, gout_ref, gloop_ref, bias_ref, bnw_ref, bnb_ref,
                  out_ref):
    lane0 = lax.broadcasted_iota(jnp.int32, (1, 16), 1) == 0
    in_res = _dinv(degi_ref[...]) * jnp.dot(
        zin_ref[...], gin_ref[...], preferred_element_type=jnp.float32, precision=lax.Precision.HIGHEST)
    out_res = _dinv(dego_ref[...]) * jnp.dot(
        zout_ref[...], gout_ref[...], preferred_element_type=jnp.float32, precision=lax.Precision.HIGHEST)
    xp = xp_ref[...]
    b = bl_ref[...]
    x0, x1, x2, x3 = (xp[:, 0:16], xp[:, 16:32], xp[:, 32:48], xp[:, 48:64])
    b0, b1, b2, b3 = (b[:, 0:16], b[:, 16:32], b[:, 32:48], b[:, 48:64])
    z22 = x2 * b2
    z0 = x0 * b0 + jnp.where(lane0, 0.0, z22)
    z1 = x1 * b1 + x3 * b3
    z2 = jnp.where(lane0, z22, x0 * b2 - x2 * b0)
    z3 = x1 * b3 - x3 * b1
    zl = jnp.concatenate([z0, z1, z2, z3], axis=1)
    loop_res = jnp.dot(zl, gloop_ref[...], preferred_element_type=jnp.float32, precision=lax.Precision.HIGHEST)
    sacc = (in_res + out_res + loop_res) * jnp.float32(1.0 / 3.0) + bias_ref[...]
    sacc = sacc * (bnw_ref[...] * jnp.float32(1.0 / np.sqrt(1.0 + 1e-5))) \
        + bnb_ref[...]
    out_ref[...] = jnp.tanh(sacc)


_RB = 5000
_GRID = NV // _RB


def _make_tc_kernels():
    params_k = pl.pallas_call(
        _params_body,
        out_shape=[
            jax.ShapeDtypeStruct((NRELX, D), jnp.float32),
            jax.ShapeDtypeStruct((D, D), jnp.float32),
            jax.ShapeDtypeStruct((D, D), jnp.float32),
            jax.ShapeDtypeStruct((D, D), jnp.float32),
            jax.ShapeDtypeStruct((NRELX, D), jnp.float32),
        ],
    )

    row_spec = pl.BlockSpec((_RB, D), lambda i: (i, 0))
    one_spec = pl.BlockSpec((_RB, 1), lambda i: (i, 0))
    mat_spec = pl.BlockSpec((D, D), lambda i: (0, 0))
    vec_spec = pl.BlockSpec((1, D), lambda i: (0, 0))

    degred_k = pl.pallas_call(
        _degred_body,
        out_shape=jax.ShapeDtypeStruct((NC, HR * 16), jnp.float32),
    )
    prep_k = pl.pallas_call(
        _prep_body,
        grid=(_GRID,),
        in_specs=[row_spec, one_spec, one_spec, mat_spec],
        out_specs=[row_spec, row_spec, row_spec],
        out_shape=[jax.ShapeDtypeStruct((NV, D), jnp.float32)] * 3,
    )

    combine_k = pl.pallas_call(
        _combine_body,
        grid=(_GRID,),
        in_specs=[row_spec, row_spec, row_spec, one_spec, one_spec, vec_spec,
                  mat_spec, mat_spec, mat_spec, vec_spec, vec_spec, vec_spec],
        out_specs=row_spec,
        out_shape=jax.ShapeDtypeStruct((NV, D), jnp.float32),
    )
    return params_k, degred_k, prep_k, combine_k


def _pad_dir(a, fill):
    return jnp.concatenate(
        [a, jnp.full((EPAD - NED,), fill, dtype=a.dtype)])


def kernel(edge_index, edge_type, init_emb, init_rel,
           w_in1, w_out1, w_loop1, w_rel1, loop_rel1, bias1, bn_w1, bn_b1,
           w_in2, w_out2, w_loop2, w_rel2, loop_rel2, bias2, bn_w2, bn_b2):
    deg_k, edge_ka, edge_kb = _make_sc_kernels()
    params_k, degred_k, prep_k, combine_k = _make_tc_kernels()
    f_mat = jnp.asarray(_F_NP)
    fi_mat = jnp.asarray(_FI_NP)

    col_in = _pad_dir(edge_index[1, :NED], 0)
    col_out = _pad_dir(edge_index[1, NED:], 0)
    row_in = _pad_dir(edge_index[0, :NED], NV)
    row_out = _pad_dir(edge_index[0, NED:], NV)
    et_in = _pad_dir(edge_type[:NED], 0)
    et_out = _pad_dir(edge_type[NED:], 0)

    def _pack(colp, rowp, etp):
        return jnp.stack([colp.reshape(NS * NCHUNK, CH),
                          rowp.reshape(NS * NCHUNK, CH),
                          etp.reshape(NS * NCHUNK, CH)], axis=1)

    ep_in = _pack(col_in, row_in, et_in)
    ep_out = _pack(col_out, row_out, et_out)

    rows2 = jnp.stack([row_in, row_out])
    deg = deg_k(rows2)
    deg = degred_k(deg.reshape(NC, NS, HR * 16))
    deg_in = deg[0, :NV].reshape(NV, 1)
    deg_out = deg[1, :NV].reshape(NV, 1)

    def layer(x, rel, loop_rel, w_in, w_out, w_loop, w_rel, bias, bn_w, bn_b):
        relx = jnp.concatenate([rel, loop_rel], axis=0)
        bt, g_in, g_out, g_loop, relw = params_k(
            relx, w_in, w_out, w_loop, w_rel, f_mat, fi_mat)
        xp, a_in, a_out = prep_k(x, deg_in, deg_out, f_mat)
        zina = edge_ka(ep_in, a_in, bt)
        a_in, zina = lax.optimization_barrier((a_in, zina))
        zinb = edge_kb(ep_in, a_in, bt)
        a_out, zinb = lax.optimization_barrier((a_out, zinb))
        zouta = edge_ka(ep_out, a_out, bt)
        a_out, zouta = lax.optimization_barrier((a_out, zouta))
        zoutb = edge_kb(ep_out, a_out, bt)
        zin = jnp.concatenate(
            [zina[0, :NV], zinb[0, :NV], zina[1, :NV], zinb[1, :NV]], axis=1)
        zout = jnp.concatenate(
            [zouta[0, :NV], zoutb[0, :NV], zouta[1, :NV], zoutb[1, :NV]],
            axis=1)
        x_next = combine_k(zin, zout, xp, deg_in, deg_out, bt[NRELX - 1:],
                           g_in, g_out, g_loop,
                           bias.reshape(1, D), bn_w.reshape(1, D),
                           bn_b.reshape(1, D))
        return x_next, relw[:NRELX - 1]

    x1, r1 = layer(init_emb, init_rel, loop_rel1,
                   w_in1, w_out1, w_loop1, w_rel1, bias1, bn_w1, bn_b1)
    x2, _ = layer(x1, r1, loop_rel2,
                  w_in2, w_out2, w_loop2, w_rel2, bias2, bn_w2, bn_b2)
    return x2


# speedup vs baseline: 1.3966x; 1.3966x over previous
import functools
import numpy as np
import jax
import jax.numpy as jnp
from jax import lax
from jax.experimental import pallas as pl
from jax.experimental.pallas import tpu as pltpu
from jax.experimental.pallas import tpu_sc as plsc

D = 64
NV = 50000
NED = 400000
NRELX = 401
NC, NS = 2, 16

CH = 128
EPT = 25088
NCHUNK = EPT // CH
EPAD = EPT * NS
ZROWS = 50048
TPR = ZROWS // NS

HR = 3200


def _build_spectral_mats():
    F = np.zeros((D, D), dtype=np.float64)
    for i in range(D):
        e = np.zeros(D)
        e[i] = 1.0
        fa = np.fft.rfft(e)
        F[i, 0:32] = fa.real[0:32]
        F[i, 32] = fa.real[32]
        F[i, 33:48] = fa.imag[1:16]
        F[i, 48:64] = fa.imag[16:32]
    Fi = np.zeros((D, D), dtype=np.float64)
    for p in range(D):
        z = np.zeros(D // 2 + 1, dtype=complex)
        if p <= 32:
            z[p] += 1.0
        else:
            z[p - 32] += 1j
        Fi[p] = np.fft.irfft(z, n=D)
    return F.astype(np.float32), Fi.astype(np.float32)


_F_NP, _FI_NP = _build_spectral_mats()
_LANE0_NP = (np.arange(16) == 0)


def _deg_body(rows_hbm, deg_out, rowv, hist_v):
    c = lax.axis_index("c")
    s = lax.axis_index("s")
    zero16 = jnp.zeros((16,), jnp.float32)

    def zero_row(i, carry):
        hist_v[i, :] = zero16
        return carry
    lax.fori_loop(0, HR, zero_row, 0)

    ccal, _ = plsc.scan_count(jnp.zeros((16,), jnp.int32))
    off = jnp.int32(16) - jnp.max(ccal)

    ebase = s * EPT

    def chunk(k, carry):
        pltpu.sync_copy(rows_hbm.at[c, pl.ds(ebase + k * CH, CH)], rowv)

        def vreg(q, carry2):
            ridx = rowv[pl.ds(q * 16, 16)]
            cnt, lastm = plsc.scan_count(ridx)
            val = (cnt + off).astype(jnp.float32)
            hi = lax.shift_right_logical(ridx, 4)
            lo = lax.bitwise_and(ridx, 15)
            plsc.addupdate_scatter(hist_v, [hi, lo], val, mask=lastm)
            return carry2
        lax.fori_loop(0, CH // 16, vreg, 0)
        return carry
    lax.fori_loop(0, NCHUNK, chunk, 0)

    pltpu.sync_copy(hist_v, deg_out.at[c, s])


def _quarter(q, arows, btv, i, t, lane0):
    if q == 0:
        a0 = arows[i, pl.ds(0, 16)]
        a2 = arows[i, pl.ds(32, 16)]
        b0 = btv[t, pl.ds(0, 16)]
        b2 = btv[t, pl.ds(32, 16)]
        return a0 * b0 + jnp.where(lane0, jnp.float32(0.0), a2 * b2)
    if q == 1:
        a1 = arows[i, pl.ds(16, 16)]
        a3 = arows[i, pl.ds(48, 16)]
        b1 = btv[t, pl.ds(16, 16)]
        b3 = btv[t, pl.ds(48, 16)]
        return a1 * b1 + a3 * b3
    if q == 2:
        a0 = arows[i, pl.ds(0, 16)]
        a2 = arows[i, pl.ds(32, 16)]
        b0 = btv[t, pl.ds(0, 16)]
        b2 = btv[t, pl.ds(32, 16)]
        return jnp.where(lane0, a2 * b2, a0 * b2 - a2 * b0)
    a1 = arows[i, pl.ds(16, 16)]
    a3 = arows[i, pl.ds(48, 16)]
    b1 = btv[t, pl.ds(16, 16)]
    b3 = btv[t, pl.ds(48, 16)]
    return a1 * b3 - a3 * b1


def _make_edge_body(q_core0, q_core1):
    def _edge_body(ep_hbm, a_hbm, b_hbm, z_out,
                   idxb, rowsb, ar0, ar1, zb0, zb1, btv, zerob, spmem_z,
                   sg0, sg1, ss0, ss1, si0, si1):
        c = lax.axis_index("c")
        s = lax.axis_index("s")
        pltpu.sync_copy(b_hbm, btv)

        zero16 = jnp.zeros((16,), jnp.float32)

        def zero_zb(i, carry):
            zerob[i, :] = zero16
            return carry
        lax.fori_loop(0, TPR // 8, zero_zb, 0)

        base = s * TPR
        for j in range(8):
            pltpu.sync_copy(zerob, spmem_z.at[pl.ds(base + j * (TPR // 8),
                                                    TPR // 8)])
        plsc.subcore_barrier()

        lane0 = lax.iota(jnp.int32, 16) == 0
        cbase = s * NCHUNK
        bufs = [(0, ar0, zb0, sg0, ss0, si0), (1, ar1, zb1, sg1, ss1, si1)]

        def compute_chunk(b, arows, zbuf):
            def group(q):
                def body(g, carry2):
                    et16 = idxb[b * 3 + 2, pl.ds(g * 16, 16)]
                    for j in range(16):
                        i = g * 16 + j
                        t = et16[j]
                        zbuf[i, :] = _quarter(q, arows, btv, i, t, lane0)
                    return carry2
                return body

            @pl.when(c == 0)
            def _():
                lax.fori_loop(0, CH // 16, group(q_core0), 0)

            @pl.when(c == 1)
            def _():
                lax.fori_loop(0, CH // 16, group(q_core1), 0)

        pltpu.sync_copy(ep_hbm.at[cbase], idxb.at[pl.ds(0, 3)])
        pltpu.async_copy(a_hbm.at[idxb.at[0]], ar0, sg0)
        pltpu.async_copy(ep_hbm.at[cbase + 1], idxb.at[pl.ds(3, 3)], si1)

        def pair(kk, carry):
            for (b, arows, zbuf, sg, ss, si) in bufs:
                k = kk * 2 + b
                nb, narows, _, nsg, _, nsi = bufs[1 - b]

                @pl.when(k >= 2)
                def _():
                    pltpu.make_async_copy(
                        zbuf, spmem_z.at[rowsb.at[b]], ss).wait()
                for g in range(CH // 16):
                    rowsb[b, pl.ds(g * 16, 16)] = \
                        idxb[b * 3 + 1, pl.ds(g * 16, 16)]

                @pl.when(k + 1 < NCHUNK)
                def _():
                    pltpu.make_async_copy(
                        ep_hbm.at[cbase], idxb.at[pl.ds(nb * 3, 3)],
                        nsi).wait()
                    pltpu.async_copy(a_hbm.at[idxb.at[nb * 3]], narows, nsg)

                pltpu.make_async_copy(a_hbm.at[pl.ds(0, CH)], arows, sg).wait()
                compute_chunk(b, arows, zbuf)
                pltpu.async_copy(
                    zbuf, spmem_z.at[rowsb.at[b]], ss, add=True)

                @pl.when(k + 2 < NCHUNK)
                def _():
                    pltpu.async_copy(ep_hbm.at[cbase + k + 2],
                                     idxb.at[pl.ds(b * 3, 3)], si)
            return carry
        lax.fori_loop(0, NCHUNK // 2, pair, 0)

        pltpu.make_async_copy(zb0, spmem_z.at[rowsb.at[0]], ss0).wait()
        pltpu.make_async_copy(zb1, spmem_z.at[rowsb.at[1]], ss1).wait()

        plsc.subcore_barrier()
        pltpu.sync_copy(spmem_z.at[pl.ds(base, TPR)],
                        z_out.at[c, pl.ds(base, TPR)])
    return _edge_body


def _make_sc_kernels():
    mesh = plsc.VectorSubcoreMesh(core_axis_name="c", subcore_axis_name="s")
    deg_k = pl.kernel(
        _deg_body,
        out_type=jax.ShapeDtypeStruct((NC, NS, HR, 16), jnp.float32),
        mesh=mesh,
        scratch_types=[
            pltpu.VMEM((CH,), jnp.int32),
            pltpu.VMEM((HR, 16), jnp.float32),
        ],
        compiler_params=pltpu.CompilerParams(
            needs_layout_passes=False, use_tc_tiling_on_sc=False),
    )
    def make_edge(q_core0, q_core1):
        return pl.kernel(
            _make_edge_body(q_core0, q_core1),
            out_type=jax.ShapeDtypeStruct((NC, ZROWS, 16), jnp.float32),
            mesh=mesh,
            scratch_types=[
                pltpu.VMEM((6, CH), jnp.int32),
                pltpu.VMEM((2, CH), jnp.int32),
                pltpu.VMEM((CH, D), jnp.float32),
                pltpu.VMEM((CH, D), jnp.float32),
                pltpu.VMEM((CH, 16), jnp.float32),
                pltpu.VMEM((CH, 16), jnp.float32),
                pltpu.VMEM((NRELX, D), jnp.float32),
                pltpu.VMEM((TPR // 8, 16), jnp.float32),
                pltpu.VMEM_SHARED((ZROWS, 16), jnp.float32),
                pltpu.SemaphoreType.DMA,
                pltpu.SemaphoreType.DMA,
                pltpu.SemaphoreType.DMA,
                pltpu.SemaphoreType.DMA,
                pltpu.SemaphoreType.DMA,
                pltpu.SemaphoreType.DMA,
            ],
            compiler_params=pltpu.CompilerParams(use_tc_tiling_on_sc=False),
        )
    return deg_k, make_edge(0, 2), make_edge(1, 3)


def _params_body(relx_ref, w_in_ref, w_out_ref, w_loop_ref, w_rel_ref,
                 f_ref, fi_ref,
                 bt_ref, gin_ref, gout_ref, gloop_ref, relw_ref):
    relx = relx_ref[...]
    fi = fi_ref[...]
    bt_ref[...] = jnp.dot(relx, f_ref[...], preferred_element_type=jnp.float32, precision=lax.Precision.HIGHEST)
    gin_ref[...] = jnp.dot(fi, w_in_ref[...], preferred_element_type=jnp.float32, precision=lax.Precision.HIGHEST)
    gout_ref[...] = jnp.dot(fi, w_out_ref[...], preferred_element_type=jnp.float32, precision=lax.Precision.HIGHEST)
    gloop_ref[...] = jnp.dot(fi, w_loop_ref[...], preferred_element_type=jnp.float32, precision=lax.Precision.HIGHEST)
    relw_ref[...] = jnp.dot(relx, w_rel_ref[...], preferred_element_type=jnp.float32, precision=lax.Precision.HIGHEST)


def _dinv(deg):
    return jnp.where(deg > 0, lax.rsqrt(jnp.maximum(deg, 1e-30)), 0.0)


def _degred_body(dt_ref, deg_ref):
    deg_ref[...] = jnp.sum(dt_ref[...], axis=1)


def _prep_body(x_ref, degi_ref, dego_ref, f_ref, xp_ref, ain_ref, aout_ref):
    xp = jnp.dot(x_ref[...], f_ref[...], preferred_element_type=jnp.float32, precision=lax.Precision.HIGHEST)
    xp_ref[...] = xp
    ain_ref[...] = _dinv(degi_ref[...]) * xp
    aout_ref[...] = _dinv(dego_ref[...]) * xp


def _combine_body(zin_ref, zout_ref, xp_ref, degi_ref, dego_ref, bl_ref,
                  gin_ref, gout_ref, gloop_ref, bias_ref, bnw_ref, bnb_ref,
                  out_ref):
    lane0 = lax.broadcasted_iota(jnp.int32, (1, 16), 1) == 0
    in_res = _dinv(degi_ref[...]) * jnp.dot(
        zin_ref[...], gin_ref[...], preferred_element_type=jnp.float32, precision=lax.Precision.HIGHEST)
    out_res = _dinv(dego_ref[...]) * jnp.dot(
        zout_ref[...], gout_ref[...], preferred_element_type=jnp.float32, precision=lax.Precision.HIGHEST)
    xp = xp_ref[...]
    b = bl_ref[...]
    x0, x1, x2, x3 = (xp[:, 0:16], xp[:, 16:32], xp[:, 32:48], xp[:, 48:64])
    b0, b1, b2, b3 = (b[:, 0:16], b[:, 16:32], b[:, 32:48], b[:, 48:64])
    z22 = x2 * b2
    z0 = x0 * b0 + jnp.where(lane0, 0.0, z22)
    z1 = x1 * b1 + x3 * b3
    z2 = jnp.where(lane0, z22, x0 * b2 - x2 * b0)
    z3 = x1 * b3 - x3 * b1
    zl = jnp.concatenate([z0, z1, z2, z3], axis=1)
    loop_res = jnp.dot(zl, gloop_ref[...], preferred_element_type=jnp.float32, precision=lax.Precision.HIGHEST)
    sacc = (in_res + out_res + loop_res) * jnp.float32(1.0 / 3.0) + bias_ref[...]
    sacc = sacc * (bnw_ref[...] * jnp.float32(1.0 / np.sqrt(1.0 + 1e-5))) \
        + bnb_ref[...]
    out_ref[...] = jnp.tanh(sacc)


_RB = 5000
_GRID = NV // _RB


def _make_tc_kernels():
    params_k = pl.pallas_call(
        _params_body,
        out_shape=[
            jax.ShapeDtypeStruct((NRELX, D), jnp.float32),
            jax.ShapeDtypeStruct((D, D), jnp.float32),
            jax.ShapeDtypeStruct((D, D), jnp.float32),
            jax.ShapeDtypeStruct((D, D), jnp.float32),
            jax.ShapeDtypeStruct((NRELX, D), jnp.float32),
        ],
    )

    row_spec = pl.BlockSpec((_RB, D), lambda i: (i, 0))
    one_spec = pl.BlockSpec((_RB, 1), lambda i: (i, 0))
    mat_spec = pl.BlockSpec((D, D), lambda i: (0, 0))
    vec_spec = pl.BlockSpec((1, D), lambda i: (0, 0))

    degred_k = pl.pallas_call(
        _degred_body,
        out_shape=jax.ShapeDtypeStruct((NC, HR * 16), jnp.float32),
    )
    prep_k = pl.pallas_call(
        _prep_body,
        grid=(_GRID,),
        in_specs=[row_spec, one_spec, one_spec, mat_spec],
        out_specs=[row_spec, row_spec, row_spec],
        out_shape=[jax.ShapeDtypeStruct((NV, D), jnp.float32)] * 3,
    )

    combine_k = pl.pallas_call(
        _combine_body,
        grid=(_GRID,),
        in_specs=[row_spec, row_spec, row_spec, one_spec, one_spec, vec_spec,
                  mat_spec, mat_spec, mat_spec, vec_spec, vec_spec, vec_spec],
        out_specs=row_spec,
        out_shape=jax.ShapeDtypeStruct((NV, D), jnp.float32),
    )
    return params_k, degred_k, prep_k, combine_k


def _pad_dir(a, fill):
    return jnp.concatenate(
        [a, jnp.full((EPAD - NED,), fill, dtype=a.dtype)])


def kernel(edge_index, edge_type, init_emb, init_rel,
           w_in1, w_out1, w_loop1, w_rel1, loop_rel1, bias1, bn_w1, bn_b1,
           w_in2, w_out2, w_loop2, w_rel2, loop_rel2, bias2, bn_w2, bn_b2):
    deg_k, edge_ka, edge_kb = _make_sc_kernels()
    params_k, degred_k, prep_k, combine_k = _make_tc_kernels()
    f_mat = jnp.asarray(_F_NP)
    fi_mat = jnp.asarray(_FI_NP)

    col_in = _pad_dir(edge_index[1, :NED], 0)
    col_out = _pad_dir(edge_index[1, NED:], 0)
    row_in = _pad_dir(edge_index[0, :NED], NV)
    row_out = _pad_dir(edge_index[0, NED:], NV)
    et_in = _pad_dir(edge_type[:NED], 0)
    et_out = _pad_dir(edge_type[NED:], 0)

    def _pack(colp, rowp, etp):
        return jnp.stack([colp.reshape(NS * NCHUNK, CH),
                          rowp.reshape(NS * NCHUNK, CH),
                          etp.reshape(NS * NCHUNK, CH)], axis=1)

    ep_in = _pack(col_in, row_in, et_in)
    ep_out = _pack(col_out, row_out, et_out)

    rows2 = jnp.stack([row_in, row_out])
    deg = deg_k(rows2)
    deg = degred_k(deg.reshape(NC, NS, HR * 16))
    deg_in = deg[0, :NV].reshape(NV, 1)
    deg_out = deg[1, :NV].reshape(NV, 1)

    def layer(x, rel, loop_rel, w_in, w_out, w_loop, w_rel, bias, bn_w, bn_b):
        relx = jnp.concatenate([rel, loop_rel], axis=0)
        bt, g_in, g_out, g_loop, relw = params_k(
            relx, w_in, w_out, w_loop, w_rel, f_mat, fi_mat)
        xp, a_in, a_out = prep_k(x, deg_in, deg_out, f_mat)
        zina = edge_ka(ep_in, a_in, bt)
        a_in, zina = lax.optimization_barrier((a_in, zina))
        zinb = edge_kb(ep_in, a_in, bt)
        a_out, zinb = lax.optimization_barrier((a_out, zinb))
        zouta = edge_ka(ep_out, a_out, bt)
        a_out, zouta = lax.optimization_barrier((a_out, zouta))
        zoutb = edge_kb(ep_out, a_out, bt)
        zin = jnp.concatenate(
            [zina[0, :NV], zinb[0, :NV], zina[1, :NV], zinb[1, :NV]], axis=1)
        zout = jnp.concatenate(
            [zouta[0, :NV], zoutb[0, :NV], zouta[1, :NV], zoutb[1, :NV]],
            axis=1)
        x_next = combine_k(zin, zout, xp, deg_in, deg_out, bt[NRELX - 1:],
                           g_in, g_out, g_loop,
                           bias.reshape(1, D), bn_w.reshape(1, D),
                           bn_b.reshape(1, D))
        return x_next, relw[:NRELX - 1]

    x1, r1 = layer(init_emb, init_rel, loop_rel1,
                   w_in1, w_out1, w_loop1, w_rel1, bias1, bn_w1, bn_b1)
    x2, _ = layer(x1, r1, loop_rel2,
                  w_in2, w_out2, w_loop2, w_rel2, bias2, bn_w2, bn_b2)
    return x2


# speedup vs baseline: 1.4800x; 1.0598x over previous
import functools
import numpy as np
import jax
import jax.numpy as jnp
from jax import lax
from jax.experimental import pallas as pl
from jax.experimental.pallas import tpu as pltpu
from jax.experimental.pallas import tpu_sc as plsc

D = 64
NV = 50000
NED = 400000
NRELX = 401
NC, NS = 2, 16

CH = 128
EPT = 25088
NCHUNK = EPT // CH
EPAD = EPT * NS
ZROWS = 50048
TPR = ZROWS // NS

HR = 3200


def _build_spectral_mats():
    F = np.zeros((D, D), dtype=np.float64)
    for i in range(D):
        e = np.zeros(D)
        e[i] = 1.0
        fa = np.fft.rfft(e)
        F[i, 0:32] = fa.real[0:32]
        F[i, 32] = fa.real[32]
        F[i, 33:48] = fa.imag[1:16]
        F[i, 48:64] = fa.imag[16:32]
    Fi = np.zeros((D, D), dtype=np.float64)
    for p in range(D):
        z = np.zeros(D // 2 + 1, dtype=complex)
        if p <= 32:
            z[p] += 1.0
        else:
            z[p - 32] += 1j
        Fi[p] = np.fft.irfft(z, n=D)
    return F.astype(np.float32), Fi.astype(np.float32)


_F_NP, _FI_NP = _build_spectral_mats()
_LANE0_NP = (np.arange(16) == 0)


def _deg_body(rows_hbm, deg_out, rowv, hist_v):
    c = lax.axis_index("c")
    s = lax.axis_index("s")
    zero16 = jnp.zeros((16,), jnp.float32)

    def zero_row(i, carry):
        hist_v[i, :] = zero16
        return carry
    lax.fori_loop(0, HR, zero_row, 0)

    ccal, _ = plsc.scan_count(jnp.zeros((16,), jnp.int32))
    off = jnp.int32(16) - jnp.max(ccal)

    ebase = s * EPT

    def chunk(k, carry):
        pltpu.sync_copy(rows_hbm.at[c, pl.ds(ebase + k * CH, CH)], rowv)

        def vreg(q, carry2):
            ridx = rowv[pl.ds(q * 16, 16)]
            cnt, lastm = plsc.scan_count(ridx)
            val = (cnt + off).astype(jnp.float32)
            hi = lax.shift_right_logical(ridx, 4)
            lo = lax.bitwise_and(ridx, 15)
            plsc.addupdate_scatter(hist_v, [hi, lo], val, mask=lastm)
            return carry2
        lax.fori_loop(0, CH // 16, vreg, 0)
        return carry
    lax.fori_loop(0, NCHUNK, chunk, 0)

    pltpu.sync_copy(hist_v, deg_out.at[c, s])


def _quarter(q, arows, btv, i, t, lane0):
    alo = arows[i, pl.ds(0, 16)]
    ahi = arows[i, pl.ds(16, 16)]
    if q == 0:
        b0 = btv[t, pl.ds(0, 16)]
        b2 = btv[t, pl.ds(32, 16)]
        return alo * b0 + jnp.where(lane0, jnp.float32(0.0), ahi * b2)
    if q == 1:
        b1 = btv[t, pl.ds(16, 16)]
        b3 = btv[t, pl.ds(48, 16)]
        return alo * b1 + ahi * b3
    if q == 2:
        b0 = btv[t, pl.ds(0, 16)]
        b2 = btv[t, pl.ds(32, 16)]
        return jnp.where(lane0, ahi * b2, alo * b2 - ahi * b0)
    b1 = btv[t, pl.ds(16, 16)]
    b3 = btv[t, pl.ds(48, 16)]
    return alo * b3 - ahi * b1


def _make_edge_body(q_core0, q_core1):
    def _edge_body(ep_hbm, a_hbm, b_hbm, z_out,
                   idxb, rowsb, ar0, ar1, zb0, zb1, btv, zerob, spmem_z,
                   sg0, sg1, ss0, ss1, si0, si1):
        c = lax.axis_index("c")
        s = lax.axis_index("s")
        pltpu.sync_copy(b_hbm, btv)

        zero16 = jnp.zeros((16,), jnp.float32)

        def zero_zb(i, carry):
            zerob[i, :] = zero16
            return carry
        lax.fori_loop(0, TPR // 8, zero_zb, 0)

        base = s * TPR
        for j in range(8):
            pltpu.sync_copy(zerob, spmem_z.at[pl.ds(base + j * (TPR // 8),
                                                    TPR // 8)])
        plsc.subcore_barrier()

        lane0 = lax.iota(jnp.int32, 16) == 0
        cbase = s * NCHUNK
        bufs = [(0, ar0, zb0, sg0, ss0, si0), (1, ar1, zb1, sg1, ss1, si1)]

        def compute_chunk(b, arows, zbuf):
            def group(q):
                def body(g, carry2):
                    et16 = idxb[b * 3 + 2, pl.ds(g * 16, 16)]
                    for j in range(16):
                        i = g * 16 + j
                        t = et16[j]
                        zbuf[i, :] = _quarter(q, arows, btv, i, t, lane0)
                    return carry2
                return body

            @pl.when(c == 0)
            def _():
                lax.fori_loop(0, CH // 16, group(q_core0), 0)

            @pl.when(c == 1)
            def _():
                lax.fori_loop(0, CH // 16, group(q_core1), 0)

        pltpu.sync_copy(ep_hbm.at[cbase], idxb.at[pl.ds(0, 3)])
        pltpu.async_copy(a_hbm.at[idxb.at[0]], ar0, sg0)
        pltpu.async_copy(ep_hbm.at[cbase + 1], idxb.at[pl.ds(3, 3)], si1)

        def pair(kk, carry):
            for (b, arows, zbuf, sg, ss, si) in bufs:
                k = kk * 2 + b
                nb, narows, _, nsg, _, nsi = bufs[1 - b]

                @pl.when(k >= 2)
                def _():
                    pltpu.make_async_copy(
                        zbuf, spmem_z.at[rowsb.at[b]], ss).wait()
                for g in range(CH // 16):
                    rowsb[b, pl.ds(g * 16, 16)] = \
                        idxb[b * 3 + 1, pl.ds(g * 16, 16)]

                @pl.when(k + 1 < NCHUNK)
                def _():
                    pltpu.make_async_copy(
                        ep_hbm.at[cbase], idxb.at[pl.ds(nb * 3, 3)],
                        nsi).wait()
                    pltpu.async_copy(a_hbm.at[idxb.at[nb * 3]], narows, nsg)

                pltpu.make_async_copy(a_hbm.at[pl.ds(0, CH)], arows, sg).wait()
                compute_chunk(b, arows, zbuf)
                pltpu.async_copy(
                    zbuf, spmem_z.at[rowsb.at[b]], ss, add=True)

                @pl.when(k + 2 < NCHUNK)
                def _():
                    pltpu.async_copy(ep_hbm.at[cbase + k + 2],
                                     idxb.at[pl.ds(b * 3, 3)], si)
            return carry
        lax.fori_loop(0, NCHUNK // 2, pair, 0)

        pltpu.make_async_copy(zb0, spmem_z.at[rowsb.at[0]], ss0).wait()
        pltpu.make_async_copy(zb1, spmem_z.at[rowsb.at[1]], ss1).wait()

        plsc.subcore_barrier()
        pltpu.sync_copy(spmem_z.at[pl.ds(base, TPR)],
                        z_out.at[c, pl.ds(base, TPR)])
    return _edge_body


def _make_sc_kernels():
    mesh = plsc.VectorSubcoreMesh(core_axis_name="c", subcore_axis_name="s")
    deg_k = pl.kernel(
        _deg_body,
        out_type=jax.ShapeDtypeStruct((NC, NS, HR, 16), jnp.float32),
        mesh=mesh,
        scratch_types=[
            pltpu.VMEM((CH,), jnp.int32),
            pltpu.VMEM((HR, 16), jnp.float32),
        ],
        compiler_params=pltpu.CompilerParams(
            needs_layout_passes=False, use_tc_tiling_on_sc=False),
    )
    def make_edge(q_core0, q_core1):
        return pl.kernel(
            _make_edge_body(q_core0, q_core1),
            out_type=jax.ShapeDtypeStruct((NC, ZROWS, 16), jnp.float32),
            mesh=mesh,
            scratch_types=[
                pltpu.VMEM((6, CH), jnp.int32),
                pltpu.VMEM((2, CH), jnp.int32),
                pltpu.VMEM((CH, 32), jnp.float32),
                pltpu.VMEM((CH, 32), jnp.float32),
                pltpu.VMEM((CH, 16), jnp.float32),
                pltpu.VMEM((CH, 16), jnp.float32),
                pltpu.VMEM((NRELX, D), jnp.float32),
                pltpu.VMEM((TPR // 8, 16), jnp.float32),
                pltpu.VMEM_SHARED((ZROWS, 16), jnp.float32),
                pltpu.SemaphoreType.DMA,
                pltpu.SemaphoreType.DMA,
                pltpu.SemaphoreType.DMA,
                pltpu.SemaphoreType.DMA,
                pltpu.SemaphoreType.DMA,
                pltpu.SemaphoreType.DMA,
            ],
            compiler_params=pltpu.CompilerParams(use_tc_tiling_on_sc=False),
        )
    return deg_k, make_edge(0, 2), make_edge(1, 3)


def _params_body(relx_ref, w_in_ref, w_out_ref, w_loop_ref, w_rel_ref,
                 f_ref, fi_ref,
                 bt_ref, gin_ref, gout_ref, gloop_ref, relw_ref):
    relx = relx_ref[...]
    fi = fi_ref[...]
    bt_ref[...] = jnp.dot(relx, f_ref[...], preferred_element_type=jnp.float32, precision=lax.Precision.HIGHEST)
    gin_ref[...] = jnp.dot(fi, w_in_ref[...], preferred_element_type=jnp.float32, precision=lax.Precision.HIGHEST)
    gout_ref[...] = jnp.dot(fi, w_out_ref[...], preferred_element_type=jnp.float32, precision=lax.Precision.HIGHEST)
    gloop_ref[...] = jnp.dot(fi, w_loop_ref[...], preferred_element_type=jnp.float32, precision=lax.Precision.HIGHEST)
    relw_ref[...] = jnp.dot(relx, w_rel_ref[...], preferred_element_type=jnp.float32, precision=lax.Precision.HIGHEST)


def _dinv(deg):
    return jnp.where(deg > 0, lax.rsqrt(jnp.maximum(deg, 1e-30)), 0.0)


def _degred_body(dt_ref, deg_ref):
    deg_ref[...] = jnp.sum(dt_ref[...], axis=1)


def _prep_body(x_ref, degi_ref, dego_ref, f_ref, xp_ref,
               aine_ref, aino_ref, aoute_ref, aouto_ref):
    xp = jnp.dot(x_ref[...], f_ref[...], preferred_element_type=jnp.float32, precision=lax.Precision.HIGHEST)
    xp_ref[...] = xp
    ain = _dinv(degi_ref[...]) * xp
    aout = _dinv(dego_ref[...]) * xp
    aine_ref[...] = jnp.concatenate([ain[:, 0:16], ain[:, 32:48]], axis=1)
    aino_ref[...] = jnp.concatenate([ain[:, 16:32], ain[:, 48:64]], axis=1)
    aoute_ref[...] = jnp.concatenate([aout[:, 0:16], aout[:, 32:48]], axis=1)
    aouto_ref[...] = jnp.concatenate([aout[:, 16:32], aout[:, 48:64]], axis=1)


def _combine_body(zin_ref, zout_ref, xp_ref, degi_ref, dego_ref, bl_ref,
                  gin_ref, gout_ref, gloop_ref, bias_ref, bnw_ref, bnb_ref,
                  out_ref):
    lane0 = lax.broadcasted_iota(jnp.int32, (1, 16), 1) == 0
    in_res = _dinv(degi_ref[...]) * jnp.dot(
        zin_ref[...], gin_ref[...], preferred_element_type=jnp.float32, precision=lax.Precision.HIGHEST)
    out_res = _dinv(dego_ref[...]) * jnp.dot(
        zout_ref[...], gout_ref[...], preferred_element_type=jnp.float32, precision=lax.Precision.HIGHEST)
    xp = xp_ref[...]
    b = bl_ref[...]
    x0, x1, x2, x3 = (xp[:, 0:16], xp[:, 16:32], xp[:, 32:48], xp[:, 48:64])
    b0, b1, b2, b3 = (b[:, 0:16], b[:, 16:32], b[:, 32:48], b[:, 48:64])
    z22 = x2 * b2
    z0 = x0 * b0 + jnp.where(lane0, 0.0, z22)
    z1 = x1 * b1 + x3 * b3
    z2 = jnp.where(lane0, z22, x0 * b2 - x2 * b0)
    z3 = x1 * b3 - x3 * b1
    zl = jnp.concatenate([z0, z1, z2, z3], axis=1)
    loop_res = jnp.dot(zl, gloop_ref[...], preferred_element_type=jnp.float32, precision=lax.Precision.HIGHEST)
    sacc = (in_res + out_res + loop_res) * jnp.float32(1.0 / 3.0) + bias_ref[...]
    sacc = sacc * (bnw_ref[...] * jnp.float32(1.0 / np.sqrt(1.0 + 1e-5))) \
        + bnb_ref[...]
    out_ref[...] = jnp.tanh(sacc)


_RB = 5000
_GRID = NV // _RB


def _make_tc_kernels():
    params_k = pl.pallas_call(
        _params_body,
        out_shape=[
            jax.ShapeDtypeStruct((NRELX, D), jnp.float32),
            jax.ShapeDtypeStruct((D, D), jnp.float32),
            jax.ShapeDtypeStruct((D, D), jnp.float32),
            jax.ShapeDtypeStruct((D, D), jnp.float32),
            jax.ShapeDtypeStruct((NRELX, D), jnp.float32),
        ],
    )

    row_spec = pl.BlockSpec((_RB, D), lambda i: (i, 0))
    one_spec = pl.BlockSpec((_RB, 1), lambda i: (i, 0))
    mat_spec = pl.BlockSpec((D, D), lambda i: (0, 0))
    vec_spec = pl.BlockSpec((1, D), lambda i: (0, 0))

    degred_k = pl.pallas_call(
        _degred_body,
        out_shape=jax.ShapeDtypeStruct((NC, HR * 16), jnp.float32),
    )
    half_spec = pl.BlockSpec((_RB, 32), lambda i: (i, 0))
    prep_k = pl.pallas_call(
        _prep_body,
        grid=(_GRID,),
        in_specs=[row_spec, one_spec, one_spec, mat_spec],
        out_specs=[row_spec, half_spec, half_spec, half_spec, half_spec],
        out_shape=[jax.ShapeDtypeStruct((NV, D), jnp.float32)]
        + [jax.ShapeDtypeStruct((NV, 32), jnp.float32)] * 4,
    )

    combine_k = pl.pallas_call(
        _combine_body,
        grid=(_GRID,),
        in_specs=[row_spec, row_spec, row_spec, one_spec, one_spec, vec_spec,
                  mat_spec, mat_spec, mat_spec, vec_spec, vec_spec, vec_spec],
        out_specs=row_spec,
        out_shape=jax.ShapeDtypeStruct((NV, D), jnp.float32),
    )
    return params_k, degred_k, prep_k, combine_k


def _pad_dir(a, fill):
    return jnp.concatenate(
        [a, jnp.full((EPAD - NED,), fill, dtype=a.dtype)])


def kernel(edge_index, edge_type, init_emb, init_rel,
           w_in1, w_out1, w_loop1, w_rel1, loop_rel1, bias1, bn_w1, bn_b1,
           w_in2, w_out2, w_loop2, w_rel2, loop_rel2, bias2, bn_w2, bn_b2):
    deg_k, edge_ka, edge_kb = _make_sc_kernels()
    params_k, degred_k, prep_k, combine_k = _make_tc_kernels()
    f_mat = jnp.asarray(_F_NP)
    fi_mat = jnp.asarray(_FI_NP)

    col_in = _pad_dir(edge_index[1, :NED], 0)
    col_out = _pad_dir(edge_index[1, NED:], 0)
    row_in = _pad_dir(edge_index[0, :NED], NV)
    row_out = _pad_dir(edge_index[0, NED:], NV)
    et_in = _pad_dir(edge_type[:NED], 0)
    et_out = _pad_dir(edge_type[NED:], 0)

    def _pack(colp, rowp, etp):
        return jnp.stack([colp.reshape(NS * NCHUNK, CH),
                          rowp.reshape(NS * NCHUNK, CH),
                          etp.reshape(NS * NCHUNK, CH)], axis=1)

    ep_in = _pack(col_in, row_in, et_in)
    ep_out = _pack(col_out, row_out, et_out)

    rows2 = jnp.stack([row_in, row_out])
    deg = deg_k(rows2)
    deg = degred_k(deg.reshape(NC, NS, HR * 16))
    deg_in = deg[0, :NV].reshape(NV, 1)
    deg_out = deg[1, :NV].reshape(NV, 1)

    def layer(x, rel, loop_rel, w_in, w_out, w_loop, w_rel, bias, bn_w, bn_b):
        relx = jnp.concatenate([rel, loop_rel], axis=0)
        bt, g_in, g_out, g_loop, relw = params_k(
            relx, w_in, w_out, w_loop, w_rel, f_mat, fi_mat)
        xp, a_in_e, a_in_o, a_out_e, a_out_o = prep_k(
            x, deg_in, deg_out, f_mat)
        zina = edge_ka(ep_in, a_in_e, bt)
        a_in_o, zina = lax.optimization_barrier((a_in_o, zina))
        zinb = edge_kb(ep_in, a_in_o, bt)
        a_out_e, zinb = lax.optimization_barrier((a_out_e, zinb))
        zouta = edge_ka(ep_out, a_out_e, bt)
        a_out_o, zouta = lax.optimization_barrier((a_out_o, zouta))
        zoutb = edge_kb(ep_out, a_out_o, bt)
        zin = jnp.concatenate(
            [zina[0, :NV], zinb[0, :NV], zina[1, :NV], zinb[1, :NV]], axis=1)
        zout = jnp.concatenate(
            [zouta[0, :NV], zoutb[0, :NV], zouta[1, :NV], zoutb[1, :NV]],
            axis=1)
        x_next = combine_k(zin, zout, xp, deg_in, deg_out, bt[NRELX - 1:],
                           g_in, g_out, g_loop,
                           bias.reshape(1, D), bn_w.reshape(1, D),
                           bn_b.reshape(1, D))
        return x_next, relw[:NRELX - 1]

    x1, r1 = layer(init_emb, init_rel, loop_rel1,
                   w_in1, w_out1, w_loop1, w_rel1, bias1, bn_w1, bn_b1)
    x2, _ = layer(x1, r1, loop_rel2,
                  w_in2, w_out2, w_loop2, w_rel2, bias2, bn_w2, bn_b2)
    return x2
